# baseline (speedup 1.0000x reference)
import functools

import jax
import jax.numpy as jnp
from jax import lax
from jax.experimental import pallas as pl
from jax.experimental.pallas import tpu as pltpu
from jax.experimental.pallas import tpu_sc as plsc

B = 4
S = 8192
D = 768
LANES = 16
NC = 2
NS = 16
NW = NC * NS
ROWS_PER_W = S // NW
K = 64
NCHUNK = ROWS_PER_W // K

BS = 1024
NSB = S // BS


def _sc_gather(tidx_hbm, pos_hbm, rows_hbm, idx_v, t0_v, buf_v, sem_g,
               sem_o):
    wid = lax.axis_index("s") * NC + lax.axis_index("c")
    base = wid * ROWS_PER_W

    pltpu.sync_copy(tidx_hbm.at[pl.ds(0, LANES)], t0_v)
    pltpu.sync_copy(tidx_hbm.at[pl.ds(base, ROWS_PER_W)], idx_v)
    t0 = lax.gather(
        t0_v[...],
        jnp.zeros((LANES, 1), jnp.int32),
        dimension_numbers=lax.GatherDimensionNumbers(
            offset_dims=(), collapsed_slice_dims=(0,), start_index_map=(0,)),
        slice_sizes=(1,),
        mode=lax.GatherScatterMode.PROMISE_IN_BOUNDS)
    for j in range(ROWS_PER_W // LANES):
        sl = pl.ds(j * LANES, LANES)
        idx_v[sl] = idx_v[sl] - t0

    def gather(ci):
        return pltpu.async_copy(
            pos_hbm.at[idx_v.at[pl.ds(ci * K, K)]], buf_v.at[ci % 2], sem_g)

    def put(ci):
        return pltpu.async_copy(
            buf_v.at[ci % 2], rows_hbm.at[pl.ds(base + ci * K, K)], sem_o)

    g = {0: gather(0)}
    o = {}
    for ci in range(NCHUNK):
        g[ci].wait()
        o[ci] = put(ci)
        if ci >= 1:
            o[ci - 1].wait()
        if ci + 1 < NCHUNK:
            g[ci + 1] = gather(ci + 1)
    o[NCHUNK - 1].wait()


def _tc_add(rows_ref, enc_ref, out_ref):
    out_ref[...] = enc_ref[...] + rows_ref[...][None]


def _tc_add_call(rows, enc_inputs):
    return pl.pallas_call(
        _tc_add,
        grid=(NSB,),
        in_specs=[
            pl.BlockSpec((BS, D), lambda s: (s, 0)),
            pl.BlockSpec((B, BS, D), lambda s: (0, s, 0)),
        ],
        out_specs=pl.BlockSpec((B, BS, D), lambda s: (0, s, 0)),
        out_shape=jax.ShapeDtypeStruct((B, S, D), jnp.float32),
    )(rows, enc_inputs)


@jax.jit
def _run(enc_inputs, tindex, pos_table):
    mesh = plsc.VectorSubcoreMesh(core_axis_name="c", subcore_axis_name="s")
    gfn = functools.partial(
        pl.kernel,
        mesh=mesh,
        out_type=jax.ShapeDtypeStruct((S, D), jnp.float32),
        scratch_types=[
            pltpu.VMEM((ROWS_PER_W,), jnp.int32),
            pltpu.VMEM((LANES,), jnp.int32),
            pltpu.VMEM((2, K, D), jnp.float32),
            pltpu.SemaphoreType.DMA,
            pltpu.SemaphoreType.DMA,
        ],
    )(_sc_gather)
    rows = gfn(tindex, pos_table)
    return _tc_add_call(rows, enc_inputs)


def kernel(enc_inputs, tindex, pos_table):
    return _run(enc_inputs, tindex, pos_table)


# speedup vs baseline: 1.0041x; 1.0041x over previous
import functools

import jax
import jax.numpy as jnp
from jax import lax
from jax.experimental import pallas as pl
from jax.experimental.pallas import tpu as pltpu
from jax.experimental.pallas import tpu_sc as plsc

B = 4
S = 8192
D = 768
LANES = 16
NC = 2
NS = 16
NW = NC * NS
ROWS_PER_W = S // NW
K = 64
NCHUNK = ROWS_PER_W // K

BS = 512
NSB = S // BS


def _sc_gather(tidx_hbm, pos_hbm, rows_hbm, idx_v, t0_v, buf_v, sem_g,
               sem_o):
    wid = lax.axis_index("s") * NC + lax.axis_index("c")
    base = wid * ROWS_PER_W

    pltpu.sync_copy(tidx_hbm.at[pl.ds(0, LANES)], t0_v)
    pltpu.sync_copy(tidx_hbm.at[pl.ds(base, ROWS_PER_W)], idx_v)
    t0 = lax.gather(
        t0_v[...],
        jnp.zeros((LANES, 1), jnp.int32),
        dimension_numbers=lax.GatherDimensionNumbers(
            offset_dims=(), collapsed_slice_dims=(0,), start_index_map=(0,)),
        slice_sizes=(1,),
        mode=lax.GatherScatterMode.PROMISE_IN_BOUNDS)
    for j in range(ROWS_PER_W // LANES):
        sl = pl.ds(j * LANES, LANES)
        idx_v[sl] = idx_v[sl] - t0

    def gather(ci):
        return pltpu.async_copy(
            pos_hbm.at[idx_v.at[pl.ds(ci * K, K)]], buf_v.at[ci % 2], sem_g)

    def put(ci):
        return pltpu.async_copy(
            buf_v.at[ci % 2], rows_hbm.at[pl.ds(base + ci * K, K)], sem_o)

    g = {0: gather(0)}
    o = {}
    for ci in range(NCHUNK):
        g[ci].wait()
        o[ci] = put(ci)
        if ci >= 1:
            o[ci - 1].wait()
        if ci + 1 < NCHUNK:
            g[ci + 1] = gather(ci + 1)
    o[NCHUNK - 1].wait()


def _tc_add(rows_ref, enc_ref, out_ref):
    out_ref[...] = enc_ref[...] + rows_ref[...][None]


def _tc_add_call(rows, enc_inputs):
    return pl.pallas_call(
        _tc_add,
        grid=(NSB,),
        in_specs=[
            pl.BlockSpec((BS, D), lambda s: (s, 0)),
            pl.BlockSpec((B, BS, D), lambda s: (0, s, 0)),
        ],
        out_specs=pl.BlockSpec((B, BS, D), lambda s: (0, s, 0)),
        out_shape=jax.ShapeDtypeStruct((B, S, D), jnp.float32),
    )(rows, enc_inputs)


@jax.jit
def _run(enc_inputs, tindex, pos_table):
    mesh = plsc.VectorSubcoreMesh(core_axis_name="c", subcore_axis_name="s")
    gfn = functools.partial(
        pl.kernel,
        mesh=mesh,
        out_type=jax.ShapeDtypeStruct((S, D), jnp.float32),
        scratch_types=[
            pltpu.VMEM((ROWS_PER_W,), jnp.int32),
            pltpu.VMEM((LANES,), jnp.int32),
            pltpu.VMEM((2, K, D), jnp.float32),
            pltpu.SemaphoreType.DMA,
            pltpu.SemaphoreType.DMA,
        ],
    )(_sc_gather)
    rows = gfn(tindex, pos_table)
    return _tc_add_call(rows, enc_inputs)


def kernel(enc_inputs, tindex, pos_table):
    return _run(enc_inputs, tindex, pos_table)
